# baseline (speedup 1.0000x reference)
import functools

import jax
import jax.numpy as jnp
from jax import lax
from jax.experimental import pallas as pl
from jax.experimental.pallas import tpu as pltpu
from jax.experimental.pallas import tpu_sc as plsc

_B = 16384
_D = 32
_NW = 32
_BPW = _B // _NW
_L = 16


def _sc_kernel(uidx_hbm, iidx_hbm, utab_hbm, itab_hbm, wvec_hbm, bvec_hbm,
               out_hbm,
               uidx_v, iidx_v, urows_v, irows_v, wv, bv, out_v,
               sem_u, sem_i):
    nc = 2
    wid = lax.axis_index("s") * nc + lax.axis_index("c")
    base = wid * _BPW

    pltpu.sync_copy(uidx_hbm.at[pl.ds(base, _BPW)], uidx_v)
    pltpu.sync_copy(iidx_hbm.at[pl.ds(base, _BPW)], iidx_v)
    cu = pltpu.async_copy(utab_hbm.at[uidx_v], urows_v, sem_u)
    ci = pltpu.async_copy(itab_hbm.at[iidx_v], irows_v, sem_i)
    pltpu.sync_copy(wvec_hbm, wv)
    pltpu.sync_copy(bvec_hbm, bv)
    cu.wait()
    ci.wait()

    wlo = wv[pl.ds(0, _L)]
    whi = wv[pl.ds(_L, _L)]
    bias = bv[pl.ds(0, _L)]
    lane = lax.iota(jnp.int32, _L)
    zero = jnp.zeros((_L,), jnp.float32)

    def group_body(g, carry):
        acc = zero
        for j in range(_L):
            r = g * _L + j
            u0 = urows_v[r, pl.ds(0, _L)]
            u1 = urows_v[r, pl.ds(_L, _L)]
            i0 = irows_v[r, pl.ds(0, _L)]
            i1 = irows_v[r, pl.ds(_L, _L)]
            s = jnp.sum(u0 * i0 * wlo + u1 * i1 * whi)
            acc = jnp.where(lane == j, jnp.full((_L,), s, jnp.float32), acc)
        acc = acc + bias
        out_v[pl.ds(g * _L, _L)] = 1.0 / (1.0 + jnp.exp(-acc))
        return carry

    lax.fori_loop(0, _BPW // _L, group_body, 0)

    pltpu.sync_copy(out_v, out_hbm.at[pl.ds(base, _BPW)])


@jax.jit
def _run(uidx, iidx, user_table, item_table, wvec, bvec):
    mesh = plsc.VectorSubcoreMesh(core_axis_name="c", subcore_axis_name="s")
    f = functools.partial(
        pl.kernel,
        out_type=jax.ShapeDtypeStruct((_B,), jnp.float32),
        mesh=mesh,
        compiler_params=pltpu.CompilerParams(
            needs_layout_passes=False, use_tc_tiling_on_sc=False),
        scratch_types=[
            pltpu.VMEM((_BPW,), jnp.int32),
            pltpu.VMEM((_BPW,), jnp.int32),
            pltpu.VMEM((_BPW, _D), jnp.float32),
            pltpu.VMEM((_BPW, _D), jnp.float32),
            pltpu.VMEM((_D,), jnp.float32),
            pltpu.VMEM((_L,), jnp.float32),
            pltpu.VMEM((_BPW,), jnp.float32),
            pltpu.SemaphoreType.DMA,
            pltpu.SemaphoreType.DMA,
        ],
    )(_sc_kernel)
    return f(uidx, iidx, user_table, item_table, wvec, bvec)


def kernel(user_indices, item_indices, user_table, item_table, W, b):
    uidx = user_indices.astype(jnp.int32)
    iidx = item_indices.astype(jnp.int32)
    wvec = W.reshape(_D).astype(jnp.float32)
    bvec = jnp.broadcast_to(b.astype(jnp.float32).reshape(1), (_L,))
    return _run(uidx, iidx, user_table, item_table, wvec, bvec)


# speedup vs baseline: 2.0313x; 2.0313x over previous
import functools

import jax
import jax.numpy as jnp
from jax import lax
from jax.experimental import pallas as pl
from jax.experimental.pallas import tpu as pltpu
from jax.experimental.pallas import tpu_sc as plsc

_B = 16384
_D = 32
_NW = 32
_BPW = _B // _NW
_L = 16
_WAVE = 8
_TC = 128


def _sc_kernel(uidx_hbm, iidx_hbm, utab_hbm, itab_hbm, wvec_hbm, bvec_hbm,
               out_hbm,
               uidx_v, iidx_v, uwave, iwave, wv, bv, out_v,
               sem_u, sem_i):
    nc = 2
    wid = lax.axis_index("s") * nc + lax.axis_index("c")
    base = wid * _BPW

    pltpu.sync_copy(uidx_hbm.at[pl.ds(base, _BPW)], uidx_v)
    pltpu.sync_copy(iidx_hbm.at[pl.ds(base, _BPW)], iidx_v)
    pltpu.sync_copy(wvec_hbm, wv)
    pltpu.sync_copy(bvec_hbm, bv)

    bias = bv[pl.ds(0, _L)]
    wlo = wv[pl.ds(0, _L)]
    whi = wv[pl.ds(_L, _L)]
    dlane = lax.iota(jnp.int32, _L)
    zero = jnp.zeros((_L,), jnp.float32)

    def sub_wave(ucol, icol, ulane, ilane, lane0, acc):
        for j0 in range(_WAVE):
            j = lane0 + j0
            cu = pl.multiple_of(ucol[j], _TC)
            ci = pl.multiple_of(icol[j], _TC)
            pltpu.async_copy(utab_hbm.at[:, pl.ds(cu, _TC)],
                             uwave.at[:, pl.ds(j0 * _TC, _TC)], sem_u)
            pltpu.async_copy(itab_hbm.at[:, pl.ds(ci, _TC)],
                             iwave.at[:, pl.ds(j0 * _TC, _TC)], sem_i)
        pltpu.make_async_copy(utab_hbm.at[:, pl.ds(0, _WAVE * _TC)],
                              uwave, sem_u).wait()
        pltpu.make_async_copy(itab_hbm.at[:, pl.ds(0, _WAVE * _TC)],
                              iwave, sem_i).wait()

        for j0 in range(_WAVE):
            j = lane0 + j0
            cu = jnp.full((_L,), j0 * _TC + ulane[j], jnp.int32)
            ci = jnp.full((_L,), j0 * _TC + ilane[j], jnp.int32)
            u0 = plsc.load_gather(uwave, [dlane, cu])
            u1 = plsc.load_gather(uwave, [dlane + _L, cu])
            v0 = plsc.load_gather(iwave, [dlane, ci])
            v1 = plsc.load_gather(iwave, [dlane + _L, ci])
            s = jnp.sum(u0 * v0 * wlo + u1 * v1 * whi)
            acc = jnp.where(dlane == j,
                            jnp.full((_L,), s, jnp.float32), acc)
        return acc

    def group_body(g, carry):
        i0 = g * _L
        uvec = uidx_v[pl.ds(i0, _L)]
        ivec = iidx_v[pl.ds(i0, _L)]
        ucol = (uvec >> 7) << 7
        icol = (ivec >> 7) << 7
        ulane = uvec & 127
        ilane = ivec & 127
        acc = sub_wave(ucol, icol, ulane, ilane, 0, zero)
        acc = sub_wave(ucol, icol, ulane, ilane, _WAVE, acc)
        acc = acc + bias
        out_v[pl.ds(i0, _L)] = 1.0 / (1.0 + jnp.exp(-acc))
        return carry

    lax.fori_loop(0, _BPW // _L, group_body, 0)

    pltpu.sync_copy(out_v, out_hbm.at[pl.ds(base, _BPW)])


@jax.jit
def _run(uidx, iidx, utab_t, itab_t, wvec, bvec):
    mesh = plsc.VectorSubcoreMesh(core_axis_name="c", subcore_axis_name="s")
    f = functools.partial(
        pl.kernel,
        out_type=jax.ShapeDtypeStruct((_B,), jnp.float32),
        mesh=mesh,
        compiler_params=pltpu.CompilerParams(
            needs_layout_passes=False, disable_bounds_checks=True),
        scratch_types=[
            pltpu.VMEM((_BPW,), jnp.int32),
            pltpu.VMEM((_BPW,), jnp.int32),
            pltpu.VMEM((_D, _WAVE * _TC), jnp.float32),
            pltpu.VMEM((_D, _WAVE * _TC), jnp.float32),
            pltpu.VMEM((_D,), jnp.float32),
            pltpu.VMEM((_L,), jnp.float32),
            pltpu.VMEM((_BPW,), jnp.float32),
            pltpu.SemaphoreType.DMA,
            pltpu.SemaphoreType.DMA,
        ],
    )(_sc_kernel)
    return f(uidx, iidx, utab_t, itab_t, wvec, bvec)


def kernel(user_indices, item_indices, user_table, item_table, W, b):
    uidx = user_indices.astype(jnp.int32)
    iidx = item_indices.astype(jnp.int32)
    utab_t = user_table.T
    itab_t = item_table.T
    wvec = W.reshape(_D).astype(jnp.float32)
    bvec = jnp.broadcast_to(b.astype(jnp.float32).reshape(1), (_L,))
    return _run(uidx, iidx, utab_t, itab_t, wvec, bvec)


# speedup vs baseline: 2.2932x; 1.1289x over previous
import functools

import jax
import jax.numpy as jnp
from jax import lax
from jax.experimental import pallas as pl
from jax.experimental.pallas import tpu as pltpu
from jax.experimental.pallas import tpu_sc as plsc

_B = 16384
_D = 32
_NW = 32
_BPW = _B // _NW
_L = 16
_WAVE = 4
_TC = 128
_NG = _BPW // _L


def _sc_kernel(uidx_hbm, iidx_hbm, utab_hbm, itab_hbm, wvec_hbm, bvec_hbm,
               out_hbm,
               uidx_v, iidx_v, ub0, ib0, ub1, ib1, wv, bv, out_v,
               su0, si0, su1, si1):
    nc = 2
    wid = lax.axis_index("s") * nc + lax.axis_index("c")
    base = wid * _BPW

    pltpu.sync_copy(uidx_hbm.at[pl.ds(base, _BPW)], uidx_v.at[pl.ds(0, _BPW)])
    pltpu.sync_copy(iidx_hbm.at[pl.ds(base, _BPW)], iidx_v.at[pl.ds(0, _BPW)])
    pltpu.sync_copy(wvec_hbm, wv)
    pltpu.sync_copy(bvec_hbm, bv)

    bias = bv[pl.ds(0, _L)]
    wlo = wv[pl.ds(0, _L)]
    whi = wv[pl.ds(_L, _L)]
    dlane = lax.iota(jnp.int32, _L)
    zero = jnp.zeros((_L,), jnp.float32)

    def fire4(ucolv, icolv, lb, pu, pi, su, si):
        for j0 in range(_WAVE):
            cu = pl.multiple_of(ucolv[lb + j0], _TC)
            ci = pl.multiple_of(icolv[lb + j0], _TC)
            pltpu.async_copy(utab_hbm.at[:, pl.ds(cu, _TC)],
                             pu.at[:, pl.ds(j0 * _TC, _TC)], su)
            pltpu.async_copy(itab_hbm.at[:, pl.ds(ci, _TC)],
                             pi.at[:, pl.ds(j0 * _TC, _TC)], si)

    def drain(pu, pi, su, si):
        pltpu.make_async_copy(utab_hbm.at[:, pl.ds(0, _WAVE * _TC)],
                              pu, su).wait()
        pltpu.make_async_copy(itab_hbm.at[:, pl.ds(0, _WAVE * _TC)],
                              pi, si).wait()

    def compute4(pu, pi, ulanev, ilanev, lb, acc):
        for j0 in range(_WAVE):
            j = lb + j0
            cu = jnp.full((_L,), j0 * _TC + ulanev[j], jnp.int32)
            ci = jnp.full((_L,), j0 * _TC + ilanev[j], jnp.int32)
            u0 = plsc.load_gather(pu, [dlane, cu])
            u1 = plsc.load_gather(pu, [dlane + _L, cu])
            v0 = plsc.load_gather(pi, [dlane, ci])
            v1 = plsc.load_gather(pi, [dlane + _L, ci])
            s = jnp.sum(u0 * v0 * wlo + u1 * v1 * whi)
            acc = jnp.where(dlane == j,
                            jnp.full((_L,), s, jnp.float32), acc)
        return acc

    def cols(vec):
        return (vec >> 7) << 7

    uvec0 = uidx_v[pl.ds(0, _L)]
    ivec0 = iidx_v[pl.ds(0, _L)]
    fire4(cols(uvec0), cols(ivec0), 0, ub0, ib0, su0, si0)

    def group_body(g, carry):
        i0 = g * _L
        uvec = uidx_v[pl.ds(i0, _L)]
        ivec = iidx_v[pl.ds(i0, _L)]
        ucol = cols(uvec)
        icol = cols(ivec)
        ulane = uvec & 127
        ilane = ivec & 127
        nuvec = uidx_v[pl.ds(i0 + _L, _L)]
        nivec = iidx_v[pl.ds(i0 + _L, _L)]
        nucol = cols(nuvec)
        nicol = cols(nivec)

        acc = zero
        fire4(ucol, icol, 4, ub1, ib1, su1, si1)
        drain(ub0, ib0, su0, si0)
        acc = compute4(ub0, ib0, ulane, ilane, 0, acc)

        fire4(ucol, icol, 8, ub0, ib0, su0, si0)
        drain(ub1, ib1, su1, si1)
        acc = compute4(ub1, ib1, ulane, ilane, 4, acc)

        fire4(ucol, icol, 12, ub1, ib1, su1, si1)
        drain(ub0, ib0, su0, si0)
        acc = compute4(ub0, ib0, ulane, ilane, 8, acc)

        @pl.when(g < _NG - 1)
        def _fire_next():
            fire4(nucol, nicol, 0, ub0, ib0, su0, si0)

        drain(ub1, ib1, su1, si1)
        acc = compute4(ub1, ib1, ulane, ilane, 12, acc)

        acc = acc + bias
        out_v[pl.ds(i0, _L)] = 1.0 / (1.0 + jnp.exp(-acc))
        return carry

    lax.fori_loop(0, _NG, group_body, 0)

    pltpu.sync_copy(out_v, out_hbm.at[pl.ds(base, _BPW)])


@jax.jit
def _run(uidx, iidx, utab_t, itab_t, wvec, bvec):
    mesh = plsc.VectorSubcoreMesh(core_axis_name="c", subcore_axis_name="s")
    f = functools.partial(
        pl.kernel,
        out_type=jax.ShapeDtypeStruct((_B,), jnp.float32),
        mesh=mesh,
        compiler_params=pltpu.CompilerParams(
            needs_layout_passes=False, disable_bounds_checks=True),
        scratch_types=[
            pltpu.VMEM((_BPW + _L,), jnp.int32),
            pltpu.VMEM((_BPW + _L,), jnp.int32),
            pltpu.VMEM((_D, _WAVE * _TC), jnp.float32),
            pltpu.VMEM((_D, _WAVE * _TC), jnp.float32),
            pltpu.VMEM((_D, _WAVE * _TC), jnp.float32),
            pltpu.VMEM((_D, _WAVE * _TC), jnp.float32),
            pltpu.VMEM((_D,), jnp.float32),
            pltpu.VMEM((_L,), jnp.float32),
            pltpu.VMEM((_BPW,), jnp.float32),
            pltpu.SemaphoreType.DMA,
            pltpu.SemaphoreType.DMA,
            pltpu.SemaphoreType.DMA,
            pltpu.SemaphoreType.DMA,
        ],
    )(_sc_kernel)
    return f(uidx, iidx, utab_t, itab_t, wvec, bvec)


def kernel(user_indices, item_indices, user_table, item_table, W, b):
    uidx = user_indices.astype(jnp.int32)
    iidx = item_indices.astype(jnp.int32)
    utab_t = user_table.T
    itab_t = item_table.T
    wvec = W.reshape(_D).astype(jnp.float32)
    bvec = jnp.broadcast_to(b.astype(jnp.float32).reshape(1), (_L,))
    return _run(uidx, iidx, utab_t, itab_t, wvec, bvec)
